# speedup vs baseline: 14.0848x; 14.0848x over previous
import jax
import jax.numpy as jnp
from jax.experimental import pallas as pl

N = 90
E = 2700
NP = 128
EP = 2720
HID = 256
OUT_C = 256
NEG = -1e30


def _gcn_layer(B, xw, b):
    deg = jnp.sum(B, axis=1, keepdims=True)
    dis = jnp.where(deg > 0, jax.lax.rsqrt(jnp.maximum(deg, 1e-12)), 0.0)
    msg = dis * jax.lax.dot(B, dis * xw)
    return jax.nn.relu(msg + b)


def _cai(h_sc, h_fc, Wb, colmask):
    P = jax.lax.dot(h_sc, Wb)
    C = jnp.tanh(jax.lax.dot_general(
        P, h_fc, (((1,), (1,)), ((), ()))))
    C_T = jnp.tanh(jax.lax.dot_general(
        h_fc, P, (((1,), (1,)), ((), ()))))
    C = jnp.where(colmask, C, NEG)
    C_T = jnp.where(colmask, C_T, NEG)
    e1 = jnp.exp(C)
    e2 = jnp.exp(C_T)
    A_sc = e1 / jnp.sum(e1, axis=1, keepdims=True)
    A_fc = e2 / jnp.sum(e2, axis=1, keepdims=True)
    cosc = jax.lax.dot(A_sc, h_fc)
    cofs = jax.lax.dot(A_fc, h_sc)
    return cosc, cofs


def _dense_net(B_sc, B_fc, x_sc, x_fc, W0, b0, W1, b1, Wb):
    colmask = jax.lax.broadcasted_iota(jnp.int32, (NP, NP), 1) < N
    xw_sc = jax.lax.dot(x_sc, W0)
    xw_fc = jax.lax.dot(x_fc, W0)
    h_sc = _gcn_layer(B_sc, xw_sc, b0)
    h_fc = _gcn_layer(B_fc, xw_fc, b0)
    cosc, cofs = _cai(h_sc, h_fc, Wb, colmask)
    x_sc1 = jnp.concatenate([h_sc, cosc], axis=1)
    x_fc1 = jnp.concatenate([h_fc, cofs], axis=1)
    h_sc2 = _gcn_layer(B_sc, jax.lax.dot(x_sc1, W1), b1)
    h_fc2 = _gcn_layer(B_fc, jax.lax.dot(x_fc1, W1), b1)
    cosc2, cofs2 = _cai(h_sc2, h_fc2, Wb, colmask)
    x_sc2 = jnp.concatenate([h_sc2, cosc2], axis=1)
    x_fc2 = jnp.concatenate([h_fc2, cofs2], axis=1)
    return x_sc1, x_sc2, x_fc1, x_fc2


def _count_matrix(dstT, src):
    row = jax.lax.broadcasted_iota(jnp.int32, (NP, EP), 0)
    col = jax.lax.broadcasted_iota(jnp.int32, (EP, NP), 1)
    D_T = (row == dstT).astype(jnp.float32)
    S = (col == src).astype(jnp.float32)
    return jax.lax.dot(D_T, S)


def _fused_body(dstT_sc, src_sc, dstT_fc, src_fc,
                x_sc, x_fc, W0, b0, W1, b1, Wb,
                o1, o2, o3, o4):
    B_sc = _count_matrix(dstT_sc[...], src_sc[...])
    B_fc = _count_matrix(dstT_fc[...], src_fc[...])
    r1, r2, r3, r4 = _dense_net(B_sc, B_fc, x_sc[...], x_fc[...],
                                W0[...], b0[...], W1[...], b1[...], Wb[...])
    o1[...] = r1
    o2[...] = r2
    o3[...] = r3
    o4[...] = r4


@jax.jit
def kernel(x_sc, x_fc, adj_sc, adj_fc, W0, b0, W1, b1, Wb):
    x_sc_p = jnp.pad(x_sc, ((0, NP - N), (0, NP - N)))
    x_fc_p = jnp.pad(x_fc, ((0, NP - N), (0, NP - N)))
    W0_p = jnp.pad(W0, ((0, NP - N), (0, 0)))
    adj_sc_p = jnp.pad(adj_sc, ((0, 0), (0, EP - E)), constant_values=NP - 1)
    adj_fc_p = jnp.pad(adj_fc, ((0, 0), (0, EP - E)), constant_values=NP - 1)
    src_sc = adj_sc_p[0].reshape(EP, 1)
    dstT_sc = adj_sc_p[1].reshape(1, EP)
    src_fc = adj_fc_p[0].reshape(EP, 1)
    dstT_fc = adj_fc_p[1].reshape(1, EP)
    out_sd = jax.ShapeDtypeStruct((NP, 2 * HID), jnp.float32)
    r1, r2, r3, r4 = pl.pallas_call(
        _fused_body,
        out_shape=(out_sd, out_sd, out_sd, out_sd),
    )(dstT_sc, src_sc, dstT_fc, src_fc,
      x_sc_p, x_fc_p, W0_p, b0.reshape(1, -1), W1, b1.reshape(1, -1), Wb)
    return r1[:N], r2[:N], r3[:N], r4[:N]


# speedup vs baseline: 67.0476x; 4.7603x over previous
import jax
import jax.numpy as jnp
from jax.experimental import pallas as pl

N = 90
E = 2700
HID = 256


def _gcn_layer(B, xw, b):
    deg = jnp.sum(B, axis=1, keepdims=True)
    dis = jnp.where(deg > 0, jax.lax.rsqrt(jnp.maximum(deg, 1e-12)), 0.0)
    msg = dis * jax.lax.dot(B, dis * xw)
    return jax.nn.relu(msg + b)


def _cai(h_sc, h_fc, Wb):
    P = jax.lax.dot(h_sc, Wb)
    C = jnp.tanh(jax.lax.dot_general(
        P, h_fc, (((1,), (1,)), ((), ()))))
    C_T = jnp.tanh(jax.lax.dot_general(
        h_fc, P, (((1,), (1,)), ((), ()))))
    e1 = jnp.exp(C)
    e2 = jnp.exp(C_T)
    A_sc = e1 / jnp.sum(e1, axis=1, keepdims=True)
    A_fc = e2 / jnp.sum(e2, axis=1, keepdims=True)
    cosc = jax.lax.dot(A_sc, h_fc)
    cofs = jax.lax.dot(A_fc, h_sc)
    return cosc, cofs


def _count_matrix(adj):
    row = jax.lax.broadcasted_iota(jnp.int32, (N, E), 0)
    D_T = (row == adj[1:2, :]).astype(jnp.float32)
    S_T = (row == adj[0:1, :]).astype(jnp.float32)
    return jax.lax.dot_general(
        D_T, S_T, (((1,), (1,)), ((), ())))


def _fused_body(adj_sc, adj_fc, x_sc, x_fc, W0, b0, W1, b1, Wb,
                o1, o2, o3, o4):
    B_sc = _count_matrix(adj_sc[...])
    B_fc = _count_matrix(adj_fc[...])
    W0v, b0v = W0[...], jnp.reshape(b0[...], (1, HID))
    W1v, b1v = W1[...], jnp.reshape(b1[...], (1, HID))
    Wbv = Wb[...]
    h_sc = _gcn_layer(B_sc, jax.lax.dot(x_sc[...], W0v), b0v)
    h_fc = _gcn_layer(B_fc, jax.lax.dot(x_fc[...], W0v), b0v)
    cosc, cofs = _cai(h_sc, h_fc, Wbv)
    x_sc1 = jnp.concatenate([h_sc, cosc], axis=1)
    x_fc1 = jnp.concatenate([h_fc, cofs], axis=1)
    h_sc2 = _gcn_layer(B_sc, jax.lax.dot(x_sc1, W1v), b1v)
    h_fc2 = _gcn_layer(B_fc, jax.lax.dot(x_fc1, W1v), b1v)
    cosc2, cofs2 = _cai(h_sc2, h_fc2, Wbv)
    o1[...] = x_sc1
    o3[...] = x_fc1
    o2[...] = jnp.concatenate([h_sc2, cosc2], axis=1)
    o4[...] = jnp.concatenate([h_fc2, cofs2], axis=1)


@jax.jit
def kernel(x_sc, x_fc, adj_sc, adj_fc, W0, b0, W1, b1, Wb):
    out_sd = jax.ShapeDtypeStruct((N, 2 * HID), jnp.float32)
    r1, r2, r3, r4 = pl.pallas_call(
        _fused_body,
        out_shape=(out_sd, out_sd, out_sd, out_sd),
    )(adj_sc, adj_fc, x_sc, x_fc, W0, b0, W1, b1, Wb)
    return r1, r2, r3, r4
